# baseline (speedup 1.0000x reference)
import functools

import jax
import jax.numpy as jnp
import numpy as np
from jax import lax
from jax.experimental import pallas as pl
from jax.experimental.pallas import tpu as pltpu
from jax.experimental.pallas import tpu_sc as plsc

D = 128
C = 500
L = 16
NCH = D // L
NC = 2
NS = 16
NW = NC * NS
BATCH = 512
RPW = BATCH // NW
F32 = jnp.float32
I32 = jnp.int32

TASK_NCOLS = (2, 3, 3, 3, 2, 3)

TASK_GATHERS = (
    ((0, 0, "e", 0), (0, 1, "e", 1)),
    ((0, 0, "e", 0), (0, 1, "e", 1), (0, 2, "e", 2)),
    ((0, 0, "e", 0), (0, 2, "e", 1), (2, 1, "e", 2), (3, 1, "e", 3),
     (1, 0, "b", 0), (1, 2, "b", 1)),
    ((0, 2, "e", 0), (2, 0, "e", 1), (1, 1, "b", 0)),
    ((0, 0, "e", 0), (0, 1, "e", 1)),
    ((0, 0, "e", 0), (0, 2, "e", 1), (2, 1, "e", 2), (3, 1, "e", 3),
     (1, 0, "b", 0), (1, 2, "b", 1)),
)


def _vsqrt(x):
    i = lax.bitcast_convert_type(x, I32)
    i = jnp.int32(0x5F3759DF) - lax.shift_right_logical(i, 1)
    r = lax.bitcast_convert_type(i, F32)
    for _ in range(3):
        r = r * (1.5 - 0.5 * x * r * r)
    return x * r


def _relu(x):
    return jnp.maximum(x, 0.0)


def _worker_id():
    return lax.axis_index("s") * NC + lax.axis_index("c")


def _ld(ref, r, c):
    if isinstance(ref, jax.Array):
        return lax.dynamic_slice(ref, (r, c * L), (1, L))[0]
    return ref[r, pl.ds(c * L, L)]


def _ldo(ref, r, c):
    if isinstance(ref, jax.Array):
        return jnp.abs(lax.dynamic_slice(ref, (r, D + c * L), (1, L))[0])
    return jnp.abs(ref[r, pl.ds(D + c * L, L)])


def _rows(body, ncarry):
    lane = lax.iota(I32, L)
    init = tuple(jnp.zeros((L,), F32) for _ in range(ncarry))

    def step(r, carry):
        accs = body(r)
        return tuple(jnp.where(lane == r, a, s)
                     for a, s in zip(accs, carry))

    res = lax.fori_loop(0, RPW, step, init)
    return res if ncarry > 1 else res[0]


def _compute_nf1(eA, eB):
    def body(r):
        acc = jnp.zeros((L,), F32)
        for c in range(NCH):
            v = _relu(jnp.abs(_ld(eA, r, c) - _ld(eB, r, c))
                      + _ldo(eA, r, c) - _ldo(eB, r, c))
            acc = acc + v * v
        return (jnp.sum(acc),)

    return jnp.sum(_rows(body, 1))


def _compute_nf2(eA, eB, eC):
    def body(r):
        aA = jnp.zeros((L,), F32)
        aB = jnp.zeros((L,), F32)
        for c in range(NCH):
            ca, oa = _ld(eA, r, c), _ldo(eA, r, c)
            cb, ob = _ld(eB, r, c), _ldo(eB, r, c)
            cc_, oc = _ld(eC, r, c), _ldo(eC, r, c)
            lo = jnp.maximum(ca - oa, cb - ob)
            hi = jnp.minimum(ca + oa, cb + ob)
            ic = (lo + hi) * 0.5
            io = jnp.abs(hi - lo) * 0.5
            v1 = _relu(jnp.abs(ic - cc_) + io - oc)
            aA = aA + v1 * v1
            v2 = _relu(lo - hi)
            aB = aB + v2 * v2
        return jnp.sum(aA), jnp.sum(aB)

    SA, SB = _rows(body, 2)
    return jnp.sum(SA + SB + 2.0 * _vsqrt(SA * SB))


def _compute_pair(eA, eB, eC, eD, bA, bB, disjoint_mode):
    def body(r):
        a1 = jnp.zeros((L,), F32)
        a2 = jnp.zeros((L,), F32)
        for c in range(NCH):
            d1 = jnp.abs(_ld(eA, r, c) + _ld(bB, r, c) - _ld(eC, r, c))
            d2 = jnp.abs(_ld(eB, r, c) + _ld(bA, r, c) - _ld(eD, r, c))
            o1, oh = _ldo(eA, r, c), _ldo(eC, r, c)
            o2, ot = _ldo(eB, r, c), _ldo(eD, r, c)
            if disjoint_mode:
                v1 = _relu(d1 - o1 - oh)
                v2 = _relu(d2 - o2 - ot)
            else:
                v1 = _relu(d1 + o1 - oh)
                v2 = _relu(d2 + o2 - ot)
            a1 = a1 + v1 * v1
            a2 = a2 + v2 * v2
        return jnp.sum(a1), jnp.sum(a2)

    S1, S2 = _rows(body, 2)
    if disjoint_mode:
        t1 = 2.0 - _vsqrt(S1)
        t2 = 2.0 - _vsqrt(S2)
        return jnp.sum(t1 * t1 + t2 * t2)
    return jnp.sum(S1 + S2 + 2.0 * _vsqrt(S1 * S2))


def _compute_nf4(eA, eB, bA):
    def body(r):
        acc = jnp.zeros((L,), F32)
        for c in range(NCH):
            v = _relu(jnp.abs(_ld(eB, r, c) - _ld(bA, r, c) - _ld(eA, r, c))
                      + _ldo(eB, r, c) - _ldo(eA, r, c))
            acc = acc + v * v
        return (jnp.sum(acc),)

    return jnp.sum(_rows(body, 1))


def _compute_dj(eA, eB):
    def body(r):
        acc = jnp.zeros((L,), F32)
        for c in range(NCH):
            v = _relu(jnp.abs(_ld(eA, r, c) - _ld(eB, r, c))
                      - _ldo(eA, r, c) - _ldo(eB, r, c))
            acc = acc + v * v
        return (jnp.sum(acc),)

    S = _rows(body, 1)
    t = _relu(2.0 - _vsqrt(S))
    return jnp.sum(t * t)


def _compute_reg(rb, wid):
    def body(r):
        acc = jnp.zeros((L,), F32)
        for c in range(NCH):
            x = _ld(rb, r, c)
            acc = acc + x * x
        m = jnp.where(wid * RPW + r < C, 1.0, 0.0)
        return (jnp.sum(acc) * m,)

    S = _rows(body, 1)
    return jnp.sum(_vsqrt(S))


def _run_task(t, ebufs, bbufs):
    if t == 0:
        return _compute_nf1(ebufs[0], ebufs[1])
    if t == 1:
        return _compute_nf2(ebufs[0], ebufs[1], ebufs[2])
    if t == 2:
        return _compute_pair(*ebufs, *bbufs, False)
    if t == 3:
        return _compute_nf4(ebufs[0], ebufs[1], bbufs[0])
    if t == 4:
        return _compute_dj(ebufs[0], ebufs[1])
    return _compute_pair(*ebufs, *bbufs, True)


def _sc_body(idsall, ce, bu, rh, rt, out,
             ibuf,
             e00, e01, e02, e03, e10, e11, e12, e13,
             b00, b01, b10, b11, rb, resbuf,
             seme0, seme1, semr):
    tables = (ce, bu, rh, rt)
    ebufs = ((e00, e01, e02, e03), (e10, e11, e12, e13))
    bbufs = ((b00, b01), (b10, b11))
    seme = (seme0, seme1)

    wid = _worker_id()
    lane = lax.iota(I32, L)

    rvec = jnp.minimum(wid * RPW + lane, C - 1)
    rcopy = pltpu.async_copy(bu.at[rvec], rb, semr)

    pltpu.sync_copy(idsall.at[pl.ds(8 * wid, 8)], ibuf)

    def fire_task(t):
        s = t % 2
        descs = []
        for (tab, col, kind, slot) in TASK_GATHERS[t]:
            dst = ebufs[s][slot] if kind == "e" else bbufs[s][slot]
            r_, o_ = divmod((t * 3 + col) * L, 128)
            descs.append(pltpu.async_copy(
                tables[tab].at[ibuf[r_, pl.ds(o_, L)]], dst, seme[s]))
        return descs

    descs = {0: fire_task(0), 1: fire_task(1)}
    partials = []
    for t in range(6):
        for c in descs[t]:
            c.wait()
        partials.append(_run_task(t, ebufs[t % 2], bbufs[t % 2]))
        if t + 2 < 6:
            descs[t + 2] = fire_task(t + 2)

    rcopy.wait()
    partials.append(_compute_reg(rb, wid))

    res = jnp.zeros((L,), F32)
    for k, p in enumerate(partials):
        res = jnp.where(lane == k, p, res)
    resbuf[...] = res
    pltpu.sync_copy(resbuf, out.at[wid])


_SCRATCH_TYPES = [
    pltpu.VMEM((8, 128), I32),
    pltpu.VMEM((RPW, 2 * D), F32),
    pltpu.VMEM((RPW, 2 * D), F32),
    pltpu.VMEM((RPW, 2 * D), F32),
    pltpu.VMEM((RPW, 2 * D), F32),
    pltpu.VMEM((RPW, 2 * D), F32),
    pltpu.VMEM((RPW, 2 * D), F32),
    pltpu.VMEM((RPW, 2 * D), F32),
    pltpu.VMEM((RPW, 2 * D), F32),
    pltpu.VMEM((RPW, D), F32),
    pltpu.VMEM((RPW, D), F32),
    pltpu.VMEM((RPW, D), F32),
    pltpu.VMEM((RPW, D), F32),
    pltpu.VMEM((RPW, D), F32),
    pltpu.VMEM((L,), F32),
    pltpu.SemaphoreType.DMA,
    pltpu.SemaphoreType.DMA,
    pltpu.SemaphoreType.DMA,
]


@functools.cache
def _get_sc_call():
    mesh = plsc.VectorSubcoreMesh(
        core_axis_name="c", subcore_axis_name="s",
        num_cores=NC, num_subcores=NS)
    return pl.kernel(
        _sc_body,
        out_type=jax.ShapeDtypeStruct((NW, L), F32),
        mesh=mesh,
        scratch_types=_SCRATCH_TYPES,
        compiler_params=pltpu.CompilerParams(needs_layout_passes=False),
    )


@functools.cache
def _sample_rows(shapes):
    with jax.ensure_compile_time_eval():
        skey = jax.random.key(7)
        rows = []
        for i, n in enumerate(shapes):
            idx = jax.random.randint(
                jax.random.fold_in(skey, i), (BATCH,), 0, n)
            rows.append(np.asarray(idx, np.int32))
    return tuple(rows)


def kernel(nf1, nf2, nf3, nf4, disjoint, nf3_neg,
           class_emb, bumps, rel_heads, rel_tails):
    pools = (nf1, nf2, nf3, nf4, disjoint, nf3_neg)
    rows = _sample_rows(tuple(p.shape[0] for p in pools))
    cols = []
    for t, p in enumerate(pools):
        s = p[rows[t]].astype(I32)
        if s.shape[1] < 3:
            s = jnp.pad(s, ((0, 0), (0, 3 - s.shape[1])))
        cols.append(s.reshape(NW, RPW, 3))
    ids4 = jnp.stack(cols, axis=1).transpose(0, 1, 3, 2)
    idsall = jnp.pad(ids4.reshape(NW, 6 * 3 * L),
                     ((0, 0), (0, 1024 - 6 * 3 * L))).reshape(NW * 8, 128)
    out = _get_sc_call()(idsall, class_emb, bumps, rel_heads, rel_tails)
    tot = jnp.sum(out, axis=0)
    loss = ((tot[0] + tot[1] + 0.25 * tot[2] + tot[3] + tot[4] + tot[5])
            / BATCH + 0.1 * tot[6] / C)
    return loss.astype(class_emb.dtype)


# speedup vs baseline: 2.4348x; 2.4348x over previous
import functools

import jax
import jax.numpy as jnp
import numpy as np
from jax import lax
from jax.experimental import pallas as pl
from jax.experimental.pallas import tpu as pltpu
from jax.experimental.pallas import tpu_sc as plsc

D = 128
C = 500
L = 16
NCH = D // L
NC = 2
NS = 16
NW = NC * NS
BATCH = 512
RPW = BATCH // NW
F32 = jnp.float32
I32 = jnp.int32

TASK_NCOLS = (2, 3, 3, 3, 2, 3)

TASK_GATHERS = (
    ((0, 0, "e", 0), (0, 1, "e", 1)),
    ((0, 0, "e", 0), (0, 1, "e", 1), (0, 2, "e", 2)),
    ((0, 0, "e", 0), (0, 2, "e", 1), (2, 1, "e", 2), (3, 1, "e", 3),
     (1, 0, "b", 0), (1, 2, "b", 1)),
    ((0, 2, "e", 0), (2, 0, "e", 1), (1, 1, "b", 0)),
    ((0, 0, "e", 0), (0, 1, "e", 1)),
    ((0, 0, "e", 0), (0, 2, "e", 1), (2, 1, "e", 2), (3, 1, "e", 3),
     (1, 0, "b", 0), (1, 2, "b", 1)),
)


def _vsqrt(x):
    i = lax.bitcast_convert_type(x, I32)
    i = jnp.int32(0x5F3759DF) - lax.shift_right_logical(i, 1)
    r = lax.bitcast_convert_type(i, F32)
    for _ in range(3):
        r = r * (1.5 - 0.5 * x * r * r)
    return x * r


def _relu(x):
    return jnp.maximum(x, 0.0)


def _worker_id():
    return lax.axis_index("s") * NC + lax.axis_index("c")


def _ld(ref, r, c):
    if isinstance(ref, jax.Array):
        return lax.dynamic_slice(ref, (r, c * L), (1, L))[0]
    return ref[r, pl.ds(c * L, L)]


def _ldo(ref, r, c):
    if isinstance(ref, jax.Array):
        return jnp.abs(lax.dynamic_slice(ref, (r, D + c * L), (1, L))[0])
    return jnp.abs(ref[r, pl.ds(D + c * L, L)])


def _rows(body, ncarry):
    lane = lax.iota(I32, L)
    init = tuple(jnp.zeros((L,), F32) for _ in range(ncarry))

    def step(r, carry):
        accs = body(r)
        return tuple(jnp.where(lane == r, a, s)
                     for a, s in zip(accs, carry))

    res = lax.fori_loop(0, RPW, step, init)
    return res if ncarry > 1 else res[0]


def _compute_nf1(eA, eB):
    def body(r):
        acc = jnp.zeros((L,), F32)
        for c in range(NCH):
            v = _relu(jnp.abs(_ld(eA, r, c) - _ld(eB, r, c))
                      + _ldo(eA, r, c) - _ldo(eB, r, c))
            acc = acc + v * v
        return (jnp.sum(acc),)

    return jnp.sum(_rows(body, 1))


def _compute_nf2(eA, eB, eC):
    def body(r):
        aA = jnp.zeros((L,), F32)
        aB = jnp.zeros((L,), F32)
        for c in range(NCH):
            ca, oa = _ld(eA, r, c), _ldo(eA, r, c)
            cb, ob = _ld(eB, r, c), _ldo(eB, r, c)
            cc_, oc = _ld(eC, r, c), _ldo(eC, r, c)
            lo = jnp.maximum(ca - oa, cb - ob)
            hi = jnp.minimum(ca + oa, cb + ob)
            ic = (lo + hi) * 0.5
            io = jnp.abs(hi - lo) * 0.5
            v1 = _relu(jnp.abs(ic - cc_) + io - oc)
            aA = aA + v1 * v1
            v2 = _relu(lo - hi)
            aB = aB + v2 * v2
        return jnp.sum(aA), jnp.sum(aB)

    SA, SB = _rows(body, 2)
    return jnp.sum(SA + SB + 2.0 * _vsqrt(SA * SB))


def _compute_pair(eA, eB, eC, eD, bA, bB, disjoint_mode):
    def body(r):
        a1 = jnp.zeros((L,), F32)
        a2 = jnp.zeros((L,), F32)
        for c in range(NCH):
            d1 = jnp.abs(_ld(eA, r, c) + _ld(bB, r, c) - _ld(eC, r, c))
            d2 = jnp.abs(_ld(eB, r, c) + _ld(bA, r, c) - _ld(eD, r, c))
            o1, oh = _ldo(eA, r, c), _ldo(eC, r, c)
            o2, ot = _ldo(eB, r, c), _ldo(eD, r, c)
            if disjoint_mode:
                v1 = _relu(d1 - o1 - oh)
                v2 = _relu(d2 - o2 - ot)
            else:
                v1 = _relu(d1 + o1 - oh)
                v2 = _relu(d2 + o2 - ot)
            a1 = a1 + v1 * v1
            a2 = a2 + v2 * v2
        return jnp.sum(a1), jnp.sum(a2)

    S1, S2 = _rows(body, 2)
    if disjoint_mode:
        t1 = 2.0 - _vsqrt(S1)
        t2 = 2.0 - _vsqrt(S2)
        return jnp.sum(t1 * t1 + t2 * t2)
    return jnp.sum(S1 + S2 + 2.0 * _vsqrt(S1 * S2))


def _compute_nf4(eA, eB, bA):
    def body(r):
        acc = jnp.zeros((L,), F32)
        for c in range(NCH):
            v = _relu(jnp.abs(_ld(eB, r, c) - _ld(bA, r, c) - _ld(eA, r, c))
                      + _ldo(eB, r, c) - _ldo(eA, r, c))
            acc = acc + v * v
        return (jnp.sum(acc),)

    return jnp.sum(_rows(body, 1))


def _compute_dj(eA, eB):
    def body(r):
        acc = jnp.zeros((L,), F32)
        for c in range(NCH):
            v = _relu(jnp.abs(_ld(eA, r, c) - _ld(eB, r, c))
                      - _ldo(eA, r, c) - _ldo(eB, r, c))
            acc = acc + v * v
        return (jnp.sum(acc),)

    S = _rows(body, 1)
    t = _relu(2.0 - _vsqrt(S))
    return jnp.sum(t * t)


def _compute_reg(rb, wid):
    def body(r):
        acc = jnp.zeros((L,), F32)
        for c in range(NCH):
            x = _ld(rb, r, c)
            acc = acc + x * x
        m = jnp.where(wid * RPW + r < C, 1.0, 0.0)
        return (jnp.sum(acc) * m,)

    S = _rows(body, 1)
    return jnp.sum(_vsqrt(S))


def _run_task(t, ebufs, bbufs):
    if t == 0:
        return _compute_nf1(ebufs[0], ebufs[1])
    if t == 1:
        return _compute_nf2(ebufs[0], ebufs[1], ebufs[2])
    if t == 2:
        return _compute_pair(*ebufs, *bbufs, False)
    if t == 3:
        return _compute_nf4(ebufs[0], ebufs[1], bbufs[0])
    if t == 4:
        return _compute_dj(ebufs[0], ebufs[1])
    return _compute_pair(*ebufs, *bbufs, True)


def _sc_body(idsall, ce, bu, rh, rt, out,
             ibuf,
             e00, e01, e02, e03, e10, e11, e12, e13,
             b00, b01, b10, b11, rb, resbuf,
             seme0, seme1, semr):
    tables = (ce, bu, rh, rt)
    ebufs = ((e00, e01, e02, e03), (e10, e11, e12, e13))
    bbufs = ((b00, b01), (b10, b11))
    seme = (seme0, seme1)

    wid = _worker_id()
    lane = lax.iota(I32, L)

    rvec = jnp.minimum(wid * RPW + lane, C - 1)
    rcopy = pltpu.async_copy(bu.at[rvec], rb, semr)

    pltpu.sync_copy(idsall.at[pl.ds(8 * wid, 8)], ibuf)

    def fire_task(t):
        s = t % 2
        descs = []
        for (tab, col, kind, slot) in TASK_GATHERS[t]:
            dst = ebufs[s][slot] if kind == "e" else bbufs[s][slot]
            r_, o_ = divmod((t * 3 + col) * L, 128)
            descs.append(pltpu.async_copy(
                tables[tab].at[ibuf[r_, pl.ds(o_, L)]], dst, seme[s]))
        return descs

    descs = {0: fire_task(0), 1: fire_task(1)}
    partials = []
    for t in range(6):
        for c in descs[t]:
            c.wait()
        partials.append(_run_task(t, ebufs[t % 2], bbufs[t % 2]))
        if t + 2 < 6:
            descs[t + 2] = fire_task(t + 2)

    rcopy.wait()
    partials.append(_compute_reg(rb, wid))

    res = jnp.zeros((L,), F32)
    for k, p in enumerate(partials):
        res = jnp.where(lane == k, p, res)
    resbuf[...] = res
    pltpu.sync_copy(resbuf, out.at[wid])


_SCRATCH_TYPES = [
    pltpu.VMEM((8, 128), I32),
    pltpu.VMEM((RPW, 2 * D), F32),
    pltpu.VMEM((RPW, 2 * D), F32),
    pltpu.VMEM((RPW, 2 * D), F32),
    pltpu.VMEM((RPW, 2 * D), F32),
    pltpu.VMEM((RPW, 2 * D), F32),
    pltpu.VMEM((RPW, 2 * D), F32),
    pltpu.VMEM((RPW, 2 * D), F32),
    pltpu.VMEM((RPW, 2 * D), F32),
    pltpu.VMEM((RPW, D), F32),
    pltpu.VMEM((RPW, D), F32),
    pltpu.VMEM((RPW, D), F32),
    pltpu.VMEM((RPW, D), F32),
    pltpu.VMEM((RPW, D), F32),
    pltpu.VMEM((L,), F32),
    pltpu.SemaphoreType.DMA,
    pltpu.SemaphoreType.DMA,
    pltpu.SemaphoreType.DMA,
]


@functools.cache
def _get_sc_call():
    mesh = plsc.VectorSubcoreMesh(
        core_axis_name="c", subcore_axis_name="s",
        num_cores=NC, num_subcores=NS)
    return pl.kernel(
        _sc_body,
        out_type=jax.ShapeDtypeStruct((NW, L), F32),
        mesh=mesh,
        scratch_types=_SCRATCH_TYPES,
        compiler_params=pltpu.CompilerParams(needs_layout_passes=False),
    )


@functools.cache
def _sample_rows(shapes):
    with jax.ensure_compile_time_eval():
        skey = jax.random.key(7)
        rows = []
        for i, n in enumerate(shapes):
            idx = jax.random.randint(
                jax.random.fold_in(skey, i), (BATCH,), 0, n)
            rows.append(np.asarray(idx, np.int32))
    return tuple(rows)


@functools.cache
def _sample_idx2d(shapes):
    rows = _sample_rows(shapes)
    out = []
    for t, _ in enumerate(shapes):
        ncols = TASK_NCOLS[t]
        r = np.broadcast_to(rows[t].reshape(NW, 1, RPW), (NW, 3, RPW))
        c = np.broadcast_to(
            np.arange(3, dtype=np.int32).reshape(1, 3, 1), (NW, 3, RPW))
        c = np.where(c < ncols, c, 0)
        out.append((np.ascontiguousarray(r), np.ascontiguousarray(c)))
    return tuple(out)


def kernel(nf1, nf2, nf3, nf4, disjoint, nf3_neg,
           class_emb, bumps, rel_heads, rel_tails):
    pools = (nf1, nf2, nf3, nf4, disjoint, nf3_neg)
    idx2d = _sample_idx2d(tuple(p.shape[0] for p in pools))
    parts = [p[jnp.asarray(r), jnp.asarray(c)].astype(I32)
             for p, (r, c) in zip(pools, idx2d)]
    idsall = jnp.pad(
        jnp.stack(parts, axis=1).reshape(NW, 6 * 3 * L),
        ((0, 0), (0, 1024 - 6 * 3 * L))).reshape(NW * 8, 128)
    out = _get_sc_call()(idsall, class_emb, bumps, rel_heads, rel_tails)
    tot = jnp.sum(out, axis=0)
    loss = ((tot[0] + tot[1] + 0.25 * tot[2] + tot[3] + tot[4] + tot[5])
            / BATCH + 0.1 * tot[6] / C)
    return loss.astype(class_emb.dtype)
